# baseline (speedup 1.0000x reference)
import functools

import jax
import jax.numpy as jnp
from jax import lax
from jax.experimental import pallas as pl
from jax.experimental.pallas import tpu as pltpu
from jax.experimental.pallas import tpu_sc as plsc

N = 10000
E = 320000
D_IN = 128
D_H = 128
D_Z = 64

NC = 2
NS = 16
NW = NC * NS
EPW = E // NW
NP_ROWS = 640

_mesh = plsc.VectorSubcoreMesh(core_axis_name="c", subcore_axis_name="s")
_sc_params = pltpu.CompilerParams(needs_layout_passes=False,
                                  use_tc_tiling_on_sc=False)


NPAD = NP_ROWS * 16


_C = 80
_NCH = EPW // _C


@functools.partial(
    pl.kernel,
    mesh=_mesh,
    out_type=jax.ShapeDtypeStruct((NW * NPAD,), jnp.float32),
    compiler_params=_sc_params,
    scratch_types=[
        pltpu.VMEM((_NCH, _C), jnp.int32),
        pltpu.VMEM((NPAD,), jnp.float32),
    ],
)
def _deg_kernel(e3_hbm, out_hbm, idxbuf, hist):
    cid = lax.axis_index("c")
    sid = lax.axis_index("s")
    wid = cid * NS + sid
    zero16 = jnp.zeros((16,), jnp.float32)
    ones16 = jnp.ones((16,), jnp.float32)

    def zh(r, carry):
        hist[pl.ds(r * 16, 16)] = zero16
        return carry

    lax.fori_loop(0, NPAD // 16, zh, 0)

    pltpu.sync_copy(e3_hbm.at[1, pl.ds(wid * _NCH, _NCH)], idxbuf)

    def hb(r, carry):
        for k in range(_C // 16):
            v = idxbuf[r, pl.ds(k * 16, 16)]
            plsc.addupdate_scatter(hist, [v], ones16)
        return carry

    lax.fori_loop(0, _NCH, hb, 0)
    pltpu.sync_copy(hist, out_hbm.at[pl.ds(wid * NPAD, NPAD)])


def _make_scatter(D, nbuf, stage_table=False):
    C, nch = _C, _NCH
    rpt = 624

    @functools.partial(
        pl.kernel,
        mesh=_mesh,
        out_type=jax.ShapeDtypeStruct((NC, N, D), jnp.float32),
        compiler_params=_sc_params,
        scratch_types=[
            pltpu.VMEM((nch, C), jnp.int32),
            pltpu.VMEM((nch, C), jnp.int32),
            pltpu.VMEM((nbuf, C, D), jnp.float32),
            pltpu.VMEM_SHARED((N, D), jnp.float32),
            [pltpu.SemaphoreType.DMA] * nbuf,
        ] + ([pltpu.VMEM_SHARED((N, D), jnp.float32)] if stage_table else []),
    )
    def _scatter(y_hbm, e3_hbm, zeros_hbm, out_hbm,
                 srcb, dstb, rows, acc, gsems, *maybe_tab):
        cid = lax.axis_index("c")
        sid = lax.axis_index("s")
        wid = cid * NS + sid
        ytab = maybe_tab[0] if stage_table else y_hbm

        pltpu.sync_copy(zeros_hbm.at[pl.ds(0, rpt)], acc.at[pl.ds(sid * rpt, rpt)])
        if stage_table:
            pltpu.sync_copy(y_hbm.at[pl.ds(sid * rpt, rpt)],
                            ytab.at[pl.ds(sid * rpt, rpt)])

        @pl.when(sid == NS - 1)
        def _init_tail():
            pltpu.sync_copy(zeros_hbm.at[pl.ds(0, N - NS * rpt)],
                            acc.at[pl.ds(NS * rpt, N - NS * rpt)])
            if stage_table:
                pltpu.sync_copy(y_hbm.at[pl.ds(NS * rpt, N - NS * rpt)],
                                ytab.at[pl.ds(NS * rpt, N - NS * rpt)])

        pltpu.sync_copy(e3_hbm.at[0, pl.ds(wid * nch, nch)], srcb)
        pltpu.sync_copy(e3_hbm.at[1, pl.ds(wid * nch, nch)], dstb)
        plsc.subcore_barrier()

        for b in range(nbuf):
            pltpu.async_copy(ytab.at[srcb.at[b]], rows.at[b], gsems[b])

        def _step(c, b):
            pltpu.make_async_copy(ytab.at[srcb.at[c]], rows.at[b],
                                  gsems[b]).wait()
            pltpu.sync_copy(rows.at[b], acc.at[dstb.at[c]], add=True)

        def body(g, carry):
            for b in range(nbuf):
                c = g * nbuf + b
                _step(c, b)

                @pl.when(c + nbuf < nch)
                def _refill():
                    pltpu.async_copy(ytab.at[srcb.at[c + nbuf]], rows.at[b],
                                     gsems[b])
            return carry

        lax.fori_loop(0, nch // nbuf, body, 0)
        for c in range((nch // nbuf) * nbuf, nch):
            _step(c, c % nbuf)

        plsc.subcore_barrier()
        pltpu.sync_copy(acc.at[pl.ds(sid * rpt, rpt)],
                        out_hbm.at[cid, pl.ds(sid * rpt, rpt)])

        @pl.when(sid == NS - 1)
        def _drain_tail():
            pltpu.sync_copy(acc.at[pl.ds(NS * rpt, N - NS * rpt)],
                            out_hbm.at[cid, pl.ds(NS * rpt, N - NS * rpt)])

    return _scatter


_scatter128 = _make_scatter(128, 3)
_scatter64 = _make_scatter(64, 4)


_R = 2000


def _t0_body(x_ref, w_ref, xw_ref):
    xw_ref[...] = jnp.dot(x_ref[...], w_ref[...],
                          preferred_element_type=jnp.float32)


def _t1_body(xw_ref, dp_ref, y_ref, d_ref):
    deg = 1.0 + jnp.sum(dp_ref[...], axis=1)
    dv = lax.rsqrt(deg)[:, None]
    y_ref[...] = dv * xw_ref[...]
    d_ref[...] = dv


def _t2_body(p_ref, y1_ref, d_ref, b_ref, w_ref, y2_ref):
    dv = d_ref[...]
    h = jnp.maximum(dv * (p_ref[0] + p_ref[1] + y1_ref[...]) + b_ref[...], 0.0)
    y2_ref[...] = dv * jnp.dot(h, w_ref[...], preferred_element_type=jnp.float32)


def _t3_body(q_ref, y2_ref, d_ref, b_ref, w1_ref, b1_ref, w2_ref, b2_ref, o_ref):
    dv = d_ref[...]
    z = dv * (q_ref[0] + q_ref[1] + y2_ref[...]) + b_ref[...]
    h2 = jnp.maximum(jnp.dot(z, w1_ref[...], preferred_element_type=jnp.float32) + b1_ref[...], 0.0)
    o_ref[...] = jnp.dot(h2, w2_ref[...], preferred_element_type=jnp.float32) + b2_ref[...]


def _rows(d):
    return pl.BlockSpec((_R, d), lambda i: (i, 0))


def _full(a, b):
    return pl.BlockSpec((a, b), lambda i: (0, 0))


def _t0(x, W):
    return pl.pallas_call(
        _t0_body,
        grid=(N // _R,),
        in_specs=[_rows(D_IN), _full(D_IN, D_H)],
        out_specs=_rows(D_H),
        out_shape=jax.ShapeDtypeStruct((N, D_H), jnp.float32),
    )(x, W)


def _t1(xw, dp):
    return pl.pallas_call(
        _t1_body,
        grid=(N // _R,),
        in_specs=[_rows(D_H),
                  pl.BlockSpec((_R, NW), lambda i: (i, 0))],
        out_specs=[_rows(D_H), _rows(1)],
        out_shape=[
            jax.ShapeDtypeStruct((N, D_H), jnp.float32),
            jax.ShapeDtypeStruct((N, 1), jnp.float32),
        ],
    )(xw, dp)


def _parts(d):
    return pl.BlockSpec((NC, _R, d), lambda i: (0, i, 0))


def _t2(p, y1, d, b, W):
    return pl.pallas_call(
        _t2_body,
        grid=(N // _R,),
        in_specs=[_parts(D_H), _rows(D_H), _rows(1),
                  _full(1, D_H), _full(D_H, D_Z)],
        out_specs=_rows(D_Z),
        out_shape=jax.ShapeDtypeStruct((N, D_Z), jnp.float32),
    )(p, y1, d, b, W)


def _t3(q, y2, d, b, W1, b1, W2, b2):
    return pl.pallas_call(
        _t3_body,
        grid=(N // _R,),
        in_specs=[_parts(D_Z), _rows(D_Z), _rows(1),
                  _full(1, D_Z), _full(D_Z, D_H), _full(1, D_H),
                  _full(D_H, D_IN), _full(1, D_IN)],
        out_specs=_rows(D_IN),
        out_shape=jax.ShapeDtypeStruct((N, D_IN), jnp.float32),
    )(q, y2, d, b, W1, b1, W2, b2)


def kernel(x, edge_index, Wc1, bc1, Wc2, bc2, Wd1, bd1, Wd2, bd2):
    e3 = edge_index.reshape(2, E // _C, _C)

    deg_parts = _deg_kernel(e3)
    dp = deg_parts.reshape(NW, NPAD).T

    xw1 = _t0(x, Wc1)
    y1, d = _t1(xw1, dp)

    p = _scatter128(y1, e3, jnp.zeros((N // NS, D_H), jnp.float32))
    y2 = _t2(p, y1, d, bc1.reshape(1, D_H), Wc2)

    q = _scatter64(y2, e3, jnp.zeros((N // NS, D_Z), jnp.float32))
    out = _t3(q, y2, d, bc2.reshape(1, D_Z), Wd1,
              bd1.reshape(1, D_H), Wd2, bd2.reshape(1, D_IN))
    return out


# speedup vs baseline: 1.0097x; 1.0097x over previous
import functools

import jax
import jax.numpy as jnp
from jax import lax
from jax.experimental import pallas as pl
from jax.experimental.pallas import tpu as pltpu
from jax.experimental.pallas import tpu_sc as plsc

N = 10000
E = 320000
D_IN = 128
D_H = 128
D_Z = 64

NC = 2
NS = 16
NW = NC * NS
EPW = E // NW
NP_ROWS = 640

_mesh = plsc.VectorSubcoreMesh(core_axis_name="c", subcore_axis_name="s")
_sc_params = pltpu.CompilerParams(needs_layout_passes=False,
                                  use_tc_tiling_on_sc=False)


NPAD = NP_ROWS * 16


_C = 80
_NCH = EPW // _C


@functools.partial(
    pl.kernel,
    mesh=_mesh,
    out_type=jax.ShapeDtypeStruct((NW * NPAD,), jnp.float32),
    compiler_params=_sc_params,
    scratch_types=[
        pltpu.VMEM((_NCH, _C), jnp.int32),
        pltpu.VMEM((NPAD,), jnp.float32),
    ],
)
def _deg_kernel(e3_hbm, out_hbm, idxbuf, hist):
    cid = lax.axis_index("c")
    sid = lax.axis_index("s")
    wid = cid * NS + sid
    zero16 = jnp.zeros((16,), jnp.float32)
    ones16 = jnp.ones((16,), jnp.float32)

    def zh(r, carry):
        hist[pl.ds(r * 16, 16)] = zero16
        return carry

    lax.fori_loop(0, NPAD // 16, zh, 0)

    pltpu.sync_copy(e3_hbm.at[1, pl.ds(wid * _NCH, _NCH)], idxbuf)

    def hb(r, carry):
        for k in range(_C // 16):
            v = idxbuf[r, pl.ds(k * 16, 16)]
            plsc.addupdate_scatter(hist, [v], ones16)
        return carry

    lax.fori_loop(0, _NCH, hb, 0)
    pltpu.sync_copy(hist, out_hbm.at[pl.ds(wid * NPAD, NPAD)])


def _make_scatter(D, nbuf, stage_table=False):
    C, nch = _C, _NCH
    rpt = 624

    @functools.partial(
        pl.kernel,
        mesh=_mesh,
        out_type=jax.ShapeDtypeStruct((NC, N, D), jnp.float32),
        compiler_params=_sc_params,
        scratch_types=[
            pltpu.VMEM((nch, C), jnp.int32),
            pltpu.VMEM((nch, C), jnp.int32),
            pltpu.VMEM((nbuf, C, D), jnp.float32),
            pltpu.VMEM_SHARED((N, D), jnp.float32),
            [pltpu.SemaphoreType.DMA] * nbuf,
        ] + ([pltpu.VMEM_SHARED((N, D), jnp.float32)] if stage_table else []),
    )
    def _scatter(y_hbm, e3_hbm, zeros_hbm, out_hbm,
                 srcb, dstb, rows, acc, gsems, *maybe_tab):
        cid = lax.axis_index("c")
        sid = lax.axis_index("s")
        wid = cid * NS + sid
        ytab = maybe_tab[0] if stage_table else y_hbm

        pltpu.sync_copy(zeros_hbm.at[pl.ds(0, rpt)], acc.at[pl.ds(sid * rpt, rpt)])
        if stage_table:
            pltpu.sync_copy(y_hbm.at[pl.ds(sid * rpt, rpt)],
                            ytab.at[pl.ds(sid * rpt, rpt)])

        @pl.when(sid == NS - 1)
        def _init_tail():
            pltpu.sync_copy(zeros_hbm.at[pl.ds(0, N - NS * rpt)],
                            acc.at[pl.ds(NS * rpt, N - NS * rpt)])
            if stage_table:
                pltpu.sync_copy(y_hbm.at[pl.ds(NS * rpt, N - NS * rpt)],
                                ytab.at[pl.ds(NS * rpt, N - NS * rpt)])

        pltpu.sync_copy(e3_hbm.at[0, pl.ds(wid * nch, nch)], srcb)
        pltpu.sync_copy(e3_hbm.at[1, pl.ds(wid * nch, nch)], dstb)
        plsc.subcore_barrier()

        for b in range(nbuf):
            pltpu.async_copy(ytab.at[srcb.at[b]], rows.at[b], gsems[b])

        def _step(c, b):
            pltpu.make_async_copy(ytab.at[srcb.at[c]], rows.at[b],
                                  gsems[b]).wait()
            pltpu.sync_copy(rows.at[b], acc.at[dstb.at[c]], add=True)

        def body(g, carry):
            for b in range(nbuf):
                c = g * nbuf + b
                _step(c, b)

                @pl.when(c + nbuf < nch)
                def _refill():
                    pltpu.async_copy(ytab.at[srcb.at[c + nbuf]], rows.at[b],
                                     gsems[b])
            return carry

        lax.fori_loop(0, nch // nbuf, body, 0)
        for c in range((nch // nbuf) * nbuf, nch):
            _step(c, c % nbuf)

        plsc.subcore_barrier()
        pltpu.sync_copy(acc.at[pl.ds(sid * rpt, rpt)],
                        out_hbm.at[cid, pl.ds(sid * rpt, rpt)])

        @pl.when(sid == NS - 1)
        def _drain_tail():
            pltpu.sync_copy(acc.at[pl.ds(NS * rpt, N - NS * rpt)],
                            out_hbm.at[cid, pl.ds(NS * rpt, N - NS * rpt)])

    return _scatter


_scatter128 = _make_scatter(128, 3)
_scatter64 = _make_scatter(64, 5)


_R = 2000


def _t1_body(x_ref, w_ref, dp_ref, y_ref, d_ref):
    deg = 1.0 + jnp.sum(dp_ref[...], axis=1)
    dv = lax.rsqrt(deg)[:, None]
    xw = jnp.dot(x_ref[...], w_ref[...], preferred_element_type=jnp.float32)
    y_ref[...] = dv * xw
    d_ref[...] = dv


def _t2_body(p_ref, y1_ref, d_ref, b_ref, w_ref, y2_ref):
    dv = d_ref[...]
    h = jnp.maximum(dv * (p_ref[0] + p_ref[1] + y1_ref[...]) + b_ref[...], 0.0)
    y2_ref[...] = dv * jnp.dot(h, w_ref[...], preferred_element_type=jnp.float32)


def _t3_body(q_ref, y2_ref, d_ref, b_ref, w1_ref, b1_ref, w2_ref, b2_ref, o_ref):
    dv = d_ref[...]
    z = dv * (q_ref[0] + q_ref[1] + y2_ref[...]) + b_ref[...]
    h2 = jnp.maximum(jnp.dot(z, w1_ref[...], preferred_element_type=jnp.float32) + b1_ref[...], 0.0)
    o_ref[...] = jnp.dot(h2, w2_ref[...], preferred_element_type=jnp.float32) + b2_ref[...]


def _rows(d):
    return pl.BlockSpec((_R, d), lambda i: (i, 0))


def _full(a, b):
    return pl.BlockSpec((a, b), lambda i: (0, 0))


def _t1(x, W, dp):
    return pl.pallas_call(
        _t1_body,
        grid=(N // _R,),
        in_specs=[_rows(D_IN), _full(D_IN, D_H),
                  pl.BlockSpec((_R, NW), lambda i: (i, 0))],
        out_specs=[_rows(D_H), _rows(1)],
        out_shape=[
            jax.ShapeDtypeStruct((N, D_H), jnp.float32),
            jax.ShapeDtypeStruct((N, 1), jnp.float32),
        ],
    )(x, W, dp)


def _parts(d):
    return pl.BlockSpec((NC, _R, d), lambda i: (0, i, 0))


def _t2(p, y1, d, b, W):
    return pl.pallas_call(
        _t2_body,
        grid=(N // _R,),
        in_specs=[_parts(D_H), _rows(D_H), _rows(1),
                  _full(1, D_H), _full(D_H, D_Z)],
        out_specs=_rows(D_Z),
        out_shape=jax.ShapeDtypeStruct((N, D_Z), jnp.float32),
    )(p, y1, d, b, W)


def _t3(q, y2, d, b, W1, b1, W2, b2):
    return pl.pallas_call(
        _t3_body,
        grid=(N // _R,),
        in_specs=[_parts(D_Z), _rows(D_Z), _rows(1),
                  _full(1, D_Z), _full(D_Z, D_H), _full(1, D_H),
                  _full(D_H, D_IN), _full(1, D_IN)],
        out_specs=_rows(D_IN),
        out_shape=jax.ShapeDtypeStruct((N, D_IN), jnp.float32),
    )(q, y2, d, b, W1, b1, W2, b2)


def kernel(x, edge_index, Wc1, bc1, Wc2, bc2, Wd1, bd1, Wd2, bd2):
    e3 = edge_index.reshape(2, E // _C, _C)

    deg_parts = _deg_kernel(e3)
    dp = deg_parts.reshape(NW, NPAD).T

    y1, d = _t1(x, Wc1, dp)

    p = _scatter128(y1, e3, jnp.zeros((N // NS, D_H), jnp.float32))
    y2 = _t2(p, y1, d, bc1.reshape(1, D_H), Wc2)

    q = _scatter64(y2, e3, jnp.zeros((N // NS, D_Z), jnp.float32))
    out = _t3(q, y2, d, bc2.reshape(1, D_Z), Wd1,
              bd1.reshape(1, D_H), Wd2, bd2.reshape(1, D_IN))
    return out
